# baseline (speedup 1.0000x reference)
import jax
import jax.numpy as jnp
from jax import lax
from jax.experimental import pallas as pl
from jax.experimental.pallas import tpu as pltpu
from jax.experimental.pallas import tpu_sc as plsc

DIM = 768
HEADS = 12
HD = DIM // HEADS
NUM_EXPERTS = 8
HIDDEN = DIM * 4
S = 2048
SBLK = 512

TOPK = 2
NSLOT = TOPK * S
BS = 256
NBLK = NSLOT // BS + NUM_EXPERTS
NROWS = NBLK * BS

NW = 32
TPW = S // NW

CUMBLK = 256


def _attn_kernel(hn_ref, wq_ref, wk_ref, wv_ref, ao_ref, k_ref, v_ref):
    h = pl.program_id(0)

    @pl.when(h == 0)
    def _init():
        hn = hn_ref[...]
        k_ref[...] = jnp.dot(hn, wk_ref[...], preferred_element_type=jnp.float32)
        v_ref[...] = jnp.dot(hn, wv_ref[...], preferred_element_type=jnp.float32)

    qh = jnp.dot(hn_ref[...], wq_ref[0], preferred_element_type=jnp.float32)
    scores = lax.dot_general(
        qh, k_ref[...], (((1,), (1,)), ((), ())),
        preferred_element_type=jnp.float32) * (1.0 / (HD ** 0.5))
    m = jnp.max(scores, axis=-1, keepdims=True)
    e = jnp.exp(scores - m)
    attnw = e / jnp.sum(e, axis=-1, keepdims=True)
    ao_ref[0] = jnp.dot(attnw, v_ref[...], preferred_element_type=jnp.float32)


def _proj_kernel(ao_ref, wo_ref, hn_ref, t_ref):
    ao2d = jnp.concatenate([ao_ref[h] for h in range(HEADS)], axis=1)
    t_ref[...] = jnp.dot(ao2d, wo_ref[...],
                         preferred_element_type=jnp.float32) + hn_ref[...]


def _plan_kernel(e1_ref, e2_ref, p1_ref, p2_ref, cnt_ref):
    lanes1 = lax.broadcasted_iota(jnp.int32, (S, NUM_EXPERTS), 1)
    oh1 = (lanes1 == e1_ref[...]).astype(jnp.float32)
    oh2 = (lanes1 == e2_ref[...]).astype(jnp.float32)
    r = lax.broadcasted_iota(jnp.int32, (CUMBLK, CUMBLK), 0)
    c = lax.broadcasted_iota(jnp.int32, (CUMBLK, CUMBLK), 1)
    ltri = (c <= r).astype(jnp.float32)

    def blocked_cumsum(oh):
        tot = jnp.zeros((1, NUM_EXPERTS), jnp.float32)
        parts = []
        for i in range(S // CUMBLK):
            chunk = oh[i * CUMBLK:(i + 1) * CUMBLK, :]
            ci = jnp.dot(ltri, chunk, preferred_element_type=jnp.float32) + tot
            tot = tot + jnp.sum(chunk, axis=0, keepdims=True)
            parts.append(ci)
        return jnp.concatenate(parts, axis=0)

    c1 = blocked_cumsum(oh1)
    c2 = blocked_cumsum(oh2)
    cnt1 = jnp.sum(oh1, axis=0, keepdims=True)
    cnt2 = jnp.sum(oh2, axis=0, keepdims=True)
    counts = cnt1 + cnt2
    nblk = jnp.floor((counts + (BS - 1)) * (1.0 / BS))
    r8 = lax.broadcasted_iota(jnp.int32, (NUM_EXPERTS, NUM_EXPERTS), 0)
    c8 = lax.broadcasted_iota(jnp.int32, (NUM_EXPERTS, NUM_EXPERTS), 1)
    strict = (r8 < c8).astype(jnp.float32)
    rowbase = jnp.dot(nblk, strict,
                      preferred_element_type=jnp.float32) * BS

    def pick(mat, oh):
        return jnp.sum(mat * oh, axis=1, keepdims=True)

    p1 = pick(rowbase + c1 - 1.0, oh1)
    p2 = pick(rowbase + cnt1 + c2 - 1.0, oh2)
    p1_ref[...] = p1.astype(jnp.int32)
    p2_ref[...] = p2.astype(jnp.int32)
    cnt_ref[...] = counts.astype(jnp.int32)


def _sc_dispatch(p12_hbm, h2_hbm, xs_hbm, idx2, rows_v, sem):
    wid = lax.axis_index("s") * 2 + lax.axis_index("c")
    tlo = wid * TPW
    pltpu.sync_copy(h2_hbm.at[pl.ds(tlo, TPW)], rows_v)
    for k in range(TOPK):
        pltpu.sync_copy(p12_hbm.at[pl.ds(k * S + tlo, TPW)], idx2.at[0])
        pltpu.async_copy(rows_v, xs_hbm.at[idx2.at[0]], sem).wait()


def _ffn_kernel(eob_ref, xs_ref, w1_ref, b1_ref, w2_ref, b2_ref, y_ref):
    xb = xs_ref[...].astype(jnp.bfloat16)
    hid = jnp.dot(xb, w1_ref[0], preferred_element_type=jnp.float32)
    hid = jax.nn.gelu(hid + b1_ref[0])
    y_ref[...] = jnp.dot(hid.astype(jnp.bfloat16), w2_ref[0],
                         preferred_element_type=jnp.float32) + b2_ref[0]


def _sc_combine(p12_hbm, ypad_hbm, y_hbm, idx_v, rows_v, sem):
    wid = lax.axis_index("s") * 2 + lax.axis_index("c")
    lo = wid * TPW
    for k in range(TOPK):
        pltpu.sync_copy(p12_hbm.at[pl.ds(k * S + lo, TPW)], idx_v)
        pltpu.async_copy(ypad_hbm.at[idx_v], rows_v, sem).wait()
        pltpu.sync_copy(rows_v, y_hbm.at[pl.ds(k * S + lo, TPW)])


def _combine_kernel(skip_ref, y1_ref, y2_ref, w1_ref, w2_ref, out_ref):
    out_ref[...] = (skip_ref[...] + w1_ref[...] * y1_ref[...]
                    + w2_ref[...] * y2_ref[...])


def _rmsnorm(x):
    n = jnp.sqrt(jnp.sum(x * x, axis=-1, keepdims=True))
    return x / jnp.maximum(n, 1e-12) * (DIM ** 0.5)


def kernel(x, Wq, Wk, Wv, Wo, Wg, W1, b1, W2, b2):
    xs = x.reshape(S, DIM)
    wq3 = Wq.reshape(DIM, HEADS, HD).transpose(1, 0, 2)

    hn = _rmsnorm(xs)

    ao = pl.pallas_call(
        _attn_kernel,
        grid=(HEADS,),
        in_specs=[
            pl.BlockSpec((S, DIM), lambda h: (0, 0)),
            pl.BlockSpec((1, DIM, HD), lambda h: (h, 0, 0)),
            pl.BlockSpec((DIM, HD), lambda h: (0, 0)),
            pl.BlockSpec((DIM, HD), lambda h: (0, 0)),
        ],
        out_specs=pl.BlockSpec((1, S, HD), lambda h: (h, 0, 0)),
        out_shape=jax.ShapeDtypeStruct((HEADS, S, HD), jnp.float32),
        scratch_shapes=[
            pltpu.VMEM((S, HD), jnp.float32),
            pltpu.VMEM((S, HD), jnp.float32),
        ],
    )(hn, wq3, Wk, Wv)

    t = pl.pallas_call(
        _proj_kernel,
        grid=(1,),
        in_specs=[
            pl.BlockSpec((HEADS, S, HD), lambda i: (0, 0, 0)),
            pl.BlockSpec((DIM, DIM), lambda i: (0, 0)),
            pl.BlockSpec((S, DIM), lambda i: (0, 0)),
        ],
        out_specs=pl.BlockSpec((S, DIM), lambda i: (0, 0)),
        out_shape=jax.ShapeDtypeStruct((S, DIM), jnp.float32),
    )(ao, Wo, hn)

    h2 = _rmsnorm(t)
    logits = h2 @ Wg
    gates = jax.nn.softmax(logits, axis=-1)
    topv, topi = jax.lax.top_k(gates, TOPK)
    denom = jnp.maximum(topv[:, 0] + topv[:, 1], 1e-9)
    w1g = (topv[:, 0] / denom)[:, None]
    w2g = (topv[:, 1] / denom)[:, None]
    e1 = topi[:, 0].astype(jnp.int32)[:, None]
    e2 = topi[:, 1].astype(jnp.int32)[:, None]

    p1, p2, counts = pl.pallas_call(
        _plan_kernel,
        grid=(1,),
        in_specs=[
            pl.BlockSpec((S, 1), lambda i: (0, 0)),
            pl.BlockSpec((S, 1), lambda i: (0, 0)),
        ],
        out_specs=[
            pl.BlockSpec((S, 1), lambda i: (0, 0)),
            pl.BlockSpec((S, 1), lambda i: (0, 0)),
            pl.BlockSpec((1, NUM_EXPERTS), lambda i: (0, 0)),
        ],
        out_shape=[
            jax.ShapeDtypeStruct((S, 1), jnp.int32),
            jax.ShapeDtypeStruct((S, 1), jnp.int32),
            jax.ShapeDtypeStruct((1, NUM_EXPERTS), jnp.int32),
        ],
    )(e1, e2)

    p12 = jnp.concatenate([p1.reshape(S), p2.reshape(S)])

    mesh = plsc.VectorSubcoreMesh(core_axis_name="c", subcore_axis_name="s",
                                  num_cores=2, num_subcores=16)
    xsrows = pl.kernel(
        _sc_dispatch,
        out_type=jax.ShapeDtypeStruct((NROWS, DIM), jnp.float32),
        mesh=mesh,
        scratch_types=[
            pltpu.VMEM((2, TPW), jnp.int32),
            pltpu.VMEM((TPW, DIM), jnp.float32),
            pltpu.SemaphoreType.DMA,
        ],
    )(p12, h2)

    nblk = (counts[0] + (BS - 1)) // BS
    eob = jnp.repeat(jnp.arange(NUM_EXPERTS, dtype=jnp.int32), nblk,
                     total_repeat_length=NBLK)

    ypad = pl.pallas_call(
        _ffn_kernel,
        grid_spec=pltpu.PrefetchScalarGridSpec(
            num_scalar_prefetch=1,
            grid=(NBLK,),
            in_specs=[
                pl.BlockSpec((BS, DIM), lambda b, eob: (b, 0)),
                pl.BlockSpec((1, DIM, HIDDEN), lambda b, eob: (eob[b], 0, 0)),
                pl.BlockSpec((1, 1, HIDDEN), lambda b, eob: (eob[b], 0, 0)),
                pl.BlockSpec((1, HIDDEN, DIM), lambda b, eob: (eob[b], 0, 0)),
                pl.BlockSpec((1, 1, DIM), lambda b, eob: (eob[b], 0, 0)),
            ],
            out_specs=pl.BlockSpec((BS, DIM), lambda b, eob: (b, 0)),
        ),
        out_shape=jax.ShapeDtypeStruct((NROWS, DIM), jnp.float32),
    )(eob, xsrows, W1.astype(jnp.bfloat16),
      b1.reshape(NUM_EXPERTS, 1, HIDDEN),
      W2.astype(jnp.bfloat16), b2.reshape(NUM_EXPERTS, 1, DIM))

    ycat = pl.kernel(
        _sc_combine,
        out_type=jax.ShapeDtypeStruct((TOPK * S, DIM), jnp.float32),
        mesh=mesh,
        scratch_types=[
            pltpu.VMEM((TPW,), jnp.int32),
            pltpu.VMEM((TPW, DIM), jnp.float32),
            pltpu.SemaphoreType.DMA,
        ],
    )(p12, ypad)
    y1 = lax.slice_in_dim(ycat, 0, S)
    y2 = lax.slice_in_dim(ycat, S, 2 * S)

    out = pl.pallas_call(
        _combine_kernel,
        grid=(S // SBLK,),
        in_specs=[
            pl.BlockSpec((SBLK, DIM), lambda i: (i, 0)),
            pl.BlockSpec((SBLK, DIM), lambda i: (i, 0)),
            pl.BlockSpec((SBLK, DIM), lambda i: (i, 0)),
            pl.BlockSpec((SBLK, 1), lambda i: (i, 0)),
            pl.BlockSpec((SBLK, 1), lambda i: (i, 0)),
        ],
        out_specs=pl.BlockSpec((SBLK, DIM), lambda i: (i, 0)),
        out_shape=jax.ShapeDtypeStruct((S, DIM), jnp.float32),
    )(xs, y1, y2, w1g, w2g)

    return out.reshape(1, S, DIM)


# speedup vs baseline: 1.0405x; 1.0405x over previous
import jax
import jax.numpy as jnp
from jax import lax
from jax.experimental import pallas as pl
from jax.experimental.pallas import tpu as pltpu
from jax.experimental.pallas import tpu_sc as plsc

DIM = 768
HEADS = 12
HD = DIM // HEADS
NUM_EXPERTS = 8
HIDDEN = DIM * 4
S = 2048
SBLK = 512

TOPK = 2
NSLOT = TOPK * S
BS = 256
NBLK = NSLOT // BS + NUM_EXPERTS
NROWS = NBLK * BS

NW = 32
TPW = S // NW

CUMBLK = 256


def _attn_kernel(hn_ref, wq_ref, wk_ref, wv_ref, wo_ref, t_ref,
                 k_ref, v_ref, ao_ref):
    h = pl.program_id(0)

    @pl.when(h == 0)
    def _init():
        hn = hn_ref[...]
        k_ref[...] = jnp.dot(hn, wk_ref[...], preferred_element_type=jnp.float32)
        v_ref[...] = jnp.dot(hn, wv_ref[...], preferred_element_type=jnp.float32)

    qh = jnp.dot(hn_ref[...], wq_ref[0], preferred_element_type=jnp.float32)
    scores = lax.dot_general(
        qh, k_ref[...], (((1,), (1,)), ((), ())),
        preferred_element_type=jnp.float32) * (1.0 / (HD ** 0.5))
    m = jnp.max(scores, axis=-1, keepdims=True)
    e = jnp.exp(scores - m)
    attnw = e * (1.0 / jnp.sum(e, axis=-1, keepdims=True))
    ao_ref[h] = jnp.dot(attnw, v_ref[...], preferred_element_type=jnp.float32)

    @pl.when(h == HEADS - 1)
    def _proj():
        ao2d = jnp.concatenate([ao_ref[i] for i in range(HEADS)], axis=1)
        t_ref[...] = jnp.dot(ao2d, wo_ref[...],
                             preferred_element_type=jnp.float32) + hn_ref[...]


def _plan_kernel(e1_ref, e2_ref, p1_ref, p2_ref, cnt_ref):
    lanes1 = lax.broadcasted_iota(jnp.int32, (S, NUM_EXPERTS), 1)
    oh1 = (lanes1 == e1_ref[...]).astype(jnp.float32)
    oh2 = (lanes1 == e2_ref[...]).astype(jnp.float32)
    r = lax.broadcasted_iota(jnp.int32, (CUMBLK, CUMBLK), 0)
    c = lax.broadcasted_iota(jnp.int32, (CUMBLK, CUMBLK), 1)
    ltri = (c <= r).astype(jnp.float32)

    def blocked_cumsum(oh):
        tot = jnp.zeros((1, NUM_EXPERTS), jnp.float32)
        parts = []
        for i in range(S // CUMBLK):
            chunk = oh[i * CUMBLK:(i + 1) * CUMBLK, :]
            ci = jnp.dot(ltri, chunk, preferred_element_type=jnp.float32) + tot
            tot = tot + jnp.sum(chunk, axis=0, keepdims=True)
            parts.append(ci)
        return jnp.concatenate(parts, axis=0)

    c1 = blocked_cumsum(oh1)
    c2 = blocked_cumsum(oh2)
    cnt1 = jnp.sum(oh1, axis=0, keepdims=True)
    cnt2 = jnp.sum(oh2, axis=0, keepdims=True)
    counts = cnt1 + cnt2
    nblk = jnp.floor((counts + (BS - 1)) * (1.0 / BS))
    r8 = lax.broadcasted_iota(jnp.int32, (NUM_EXPERTS, NUM_EXPERTS), 0)
    c8 = lax.broadcasted_iota(jnp.int32, (NUM_EXPERTS, NUM_EXPERTS), 1)
    strict = (r8 < c8).astype(jnp.float32)
    rowbase = jnp.dot(nblk, strict,
                      preferred_element_type=jnp.float32) * BS

    def pick(mat, oh):
        return jnp.sum(mat * oh, axis=1, keepdims=True)

    p1 = pick(rowbase + c1 - 1.0, oh1)
    p2 = pick(rowbase + cnt1 + c2 - 1.0, oh2)
    p1_ref[...] = p1.astype(jnp.int32)
    p2_ref[...] = p2.astype(jnp.int32)
    cnt_ref[...] = counts.astype(jnp.int32)


def _sc_dispatch(p12_hbm, h2_hbm, xs_hbm, idx2, rows_v, sem):
    wid = lax.axis_index("s") * 2 + lax.axis_index("c")
    tlo = wid * TPW
    pltpu.sync_copy(h2_hbm.at[pl.ds(tlo, TPW)], rows_v)
    for k in range(TOPK):
        pltpu.sync_copy(p12_hbm.at[pl.ds(k * S + tlo, TPW)], idx2.at[0])
        pltpu.async_copy(rows_v, xs_hbm.at[idx2.at[0]], sem).wait()


def _ffn_kernel(eob_ref, xs_ref, w1_ref, b1_ref, w2_ref, b2_ref, y_ref):
    xb = xs_ref[...].astype(jnp.bfloat16)
    hid = jnp.dot(xb, w1_ref[0], preferred_element_type=jnp.float32)
    hid = jax.nn.gelu(hid + b1_ref[0])
    y_ref[...] = jnp.dot(hid.astype(jnp.bfloat16), w2_ref[0],
                         preferred_element_type=jnp.float32) + b2_ref[0]


def _sc_combine(p12_hbm, ypad_hbm, y_hbm, idx_v, rows_v, sem):
    wid = lax.axis_index("s") * 2 + lax.axis_index("c")
    lo = wid * TPW
    for k in range(TOPK):
        pltpu.sync_copy(p12_hbm.at[pl.ds(k * S + lo, TPW)], idx_v)
        pltpu.async_copy(ypad_hbm.at[idx_v], rows_v, sem).wait()
        pltpu.sync_copy(rows_v, y_hbm.at[pl.ds(k * S + lo, TPW)])


def _combine_kernel(skip_ref, y1_ref, y2_ref, w1_ref, w2_ref, out_ref):
    out_ref[...] = (skip_ref[...] + w1_ref[...] * y1_ref[...]
                    + w2_ref[...] * y2_ref[...])


def _rmsnorm(x):
    n = jnp.sqrt(jnp.sum(x * x, axis=-1, keepdims=True))
    return x / jnp.maximum(n, 1e-12) * (DIM ** 0.5)


def kernel(x, Wq, Wk, Wv, Wo, Wg, W1, b1, W2, b2):
    xs = x.reshape(S, DIM)
    wq3 = Wq.reshape(DIM, HEADS, HD).transpose(1, 0, 2)

    hn = _rmsnorm(xs)

    t = pl.pallas_call(
        _attn_kernel,
        grid=(HEADS,),
        in_specs=[
            pl.BlockSpec((S, DIM), lambda h: (0, 0)),
            pl.BlockSpec((1, DIM, HD), lambda h: (h, 0, 0)),
            pl.BlockSpec((DIM, HD), lambda h: (0, 0)),
            pl.BlockSpec((DIM, HD), lambda h: (0, 0)),
            pl.BlockSpec((DIM, DIM), lambda h: (0, 0)),
        ],
        out_specs=pl.BlockSpec((S, DIM), lambda h: (0, 0)),
        out_shape=jax.ShapeDtypeStruct((S, DIM), jnp.float32),
        scratch_shapes=[
            pltpu.VMEM((S, HD), jnp.float32),
            pltpu.VMEM((S, HD), jnp.float32),
            pltpu.VMEM((HEADS, S, HD), jnp.float32),
        ],
    )(hn, wq3, Wk, Wv, Wo)

    h2 = _rmsnorm(t)
    logits = h2 @ Wg
    gates = jax.nn.softmax(logits, axis=-1)
    topv, topi = jax.lax.top_k(gates, TOPK)
    denom = jnp.maximum(topv[:, 0] + topv[:, 1], 1e-9)
    w1g = (topv[:, 0] / denom)[:, None]
    w2g = (topv[:, 1] / denom)[:, None]
    e1 = topi[:, 0].astype(jnp.int32)[:, None]
    e2 = topi[:, 1].astype(jnp.int32)[:, None]

    p1, p2, counts = pl.pallas_call(
        _plan_kernel,
        grid=(1,),
        in_specs=[
            pl.BlockSpec((S, 1), lambda i: (0, 0)),
            pl.BlockSpec((S, 1), lambda i: (0, 0)),
        ],
        out_specs=[
            pl.BlockSpec((S, 1), lambda i: (0, 0)),
            pl.BlockSpec((S, 1), lambda i: (0, 0)),
            pl.BlockSpec((1, NUM_EXPERTS), lambda i: (0, 0)),
        ],
        out_shape=[
            jax.ShapeDtypeStruct((S, 1), jnp.int32),
            jax.ShapeDtypeStruct((S, 1), jnp.int32),
            jax.ShapeDtypeStruct((1, NUM_EXPERTS), jnp.int32),
        ],
    )(e1, e2)

    p12 = jnp.concatenate([p1.reshape(S), p2.reshape(S)])

    mesh = plsc.VectorSubcoreMesh(core_axis_name="c", subcore_axis_name="s",
                                  num_cores=2, num_subcores=16)
    xsrows = pl.kernel(
        _sc_dispatch,
        out_type=jax.ShapeDtypeStruct((NROWS, DIM), jnp.float32),
        mesh=mesh,
        scratch_types=[
            pltpu.VMEM((2, TPW), jnp.int32),
            pltpu.VMEM((TPW, DIM), jnp.float32),
            pltpu.SemaphoreType.DMA,
        ],
    )(p12, h2)

    nblk = (counts[0] + (BS - 1)) // BS
    eob = jnp.repeat(jnp.arange(NUM_EXPERTS, dtype=jnp.int32), nblk,
                     total_repeat_length=NBLK)

    ypad = pl.pallas_call(
        _ffn_kernel,
        grid_spec=pltpu.PrefetchScalarGridSpec(
            num_scalar_prefetch=1,
            grid=(NBLK,),
            in_specs=[
                pl.BlockSpec((BS, DIM), lambda b, eob: (b, 0)),
                pl.BlockSpec((1, DIM, HIDDEN), lambda b, eob: (eob[b], 0, 0)),
                pl.BlockSpec((1, 1, HIDDEN), lambda b, eob: (eob[b], 0, 0)),
                pl.BlockSpec((1, HIDDEN, DIM), lambda b, eob: (eob[b], 0, 0)),
                pl.BlockSpec((1, 1, DIM), lambda b, eob: (eob[b], 0, 0)),
            ],
            out_specs=pl.BlockSpec((BS, DIM), lambda b, eob: (b, 0)),
        ),
        out_shape=jax.ShapeDtypeStruct((NROWS, DIM), jnp.float32),
    )(eob, xsrows, W1.astype(jnp.bfloat16),
      b1.reshape(NUM_EXPERTS, 1, HIDDEN),
      W2.astype(jnp.bfloat16), b2.reshape(NUM_EXPERTS, 1, DIM))

    ycat = pl.kernel(
        _sc_combine,
        out_type=jax.ShapeDtypeStruct((TOPK * S, DIM), jnp.float32),
        mesh=mesh,
        scratch_types=[
            pltpu.VMEM((TPW,), jnp.int32),
            pltpu.VMEM((TPW, DIM), jnp.float32),
            pltpu.SemaphoreType.DMA,
        ],
    )(p12, ypad)

    nsb = S // SBLK
    out = pl.pallas_call(
        _combine_kernel,
        grid=(nsb,),
        in_specs=[
            pl.BlockSpec((SBLK, DIM), lambda i: (i, 0)),
            pl.BlockSpec((SBLK, DIM), lambda i: (i, 0)),
            pl.BlockSpec((SBLK, DIM), lambda i: (nsb + i, 0)),
            pl.BlockSpec((SBLK, 1), lambda i: (i, 0)),
            pl.BlockSpec((SBLK, 1), lambda i: (i, 0)),
        ],
        out_specs=pl.BlockSpec((SBLK, DIM), lambda i: (i, 0)),
        out_shape=jax.ShapeDtypeStruct((S, DIM), jnp.float32),
    )(xs, ycat, ycat, w1g, w2g)

    return out.reshape(1, S, DIM)
